# baseline (speedup 1.0000x reference)
import jax
import jax.numpy as jnp
from jax import lax
from jax.experimental import pallas as pl
from jax.experimental.pallas import tpu as pltpu
from jax.experimental.pallas import tpu_sc as plsc

N_TOK = 64
D = 768
E = 64
H = 768

_F32 = jnp.float32


def _router_body(x_ref, rw_ref, gi01_ref, gi0_ref, gi1_ref, gw0_ref, gw1_ref,
                 aux_ref):
    x = x_ref[...]
    rw = rw_ref[...]
    logits = lax.dot_general(x, rw, (((1,), (1,)), ((), ())),
                             preferred_element_type=_F32)
    m = jnp.max(logits, axis=1, keepdims=True)
    el = jnp.exp(logits - m)
    probs = el / jnp.sum(el, axis=1, keepdims=True)

    lane = lax.broadcasted_iota(jnp.int32, (N_TOK, E), 1)
    v0 = jnp.max(probs, axis=1, keepdims=True)
    i0 = jnp.min(jnp.where(probs == v0, lane, E), axis=1, keepdims=True)
    probs2 = jnp.where(lane == i0, -1.0, probs)
    v1 = jnp.max(probs2, axis=1, keepdims=True)
    i1 = jnp.min(jnp.where(probs2 == v1, lane, E), axis=1, keepdims=True)

    s = v0 + v1
    gi01_ref[0:N_TOK] = i0
    gi01_ref[N_TOK:2 * N_TOK] = i1
    gi0_ref[...] = i0
    gi1_ref[...] = i1
    gw0_ref[...] = v0 / s
    gw1_ref[...] = v1 / s

    one0 = (lane == i0).astype(_F32)
    f = jnp.sum(one0, axis=0, keepdims=True) / N_TOK
    P = jnp.sum(probs, axis=0, keepdims=True) / N_TOK
    aux_ref[...] = jnp.sum(f * P, keepdims=True) * E


def _router(x2d, router_w):
    return pl.pallas_call(
        _router_body,
        out_shape=(
            jax.ShapeDtypeStruct((2 * N_TOK, 1), jnp.int32),
            jax.ShapeDtypeStruct((N_TOK, 1), jnp.int32),
            jax.ShapeDtypeStruct((N_TOK, 1), jnp.int32),
            jax.ShapeDtypeStruct((N_TOK, 1), _F32),
            jax.ShapeDtypeStruct((N_TOK, 1), _F32),
            jax.ShapeDtypeStruct((1, 1), _F32),
        ),
    )(x2d, router_w)


_L = 16
_NV = E // _L
_OUTW = E + _L
K_SPEC = 16


def _dispatch_body(gi01_hbm, out_hbm, idx_v, flag_v, ids_v, sem):
    cid = lax.axis_index("c")
    sid = lax.axis_index("s")

    @pl.when((cid == 0) & (sid == 0))
    def _():
        pltpu.sync_copy(gi01_hbm, idx_v)
        zeros = jnp.zeros((_L,), jnp.int32)
        ones = jnp.ones((_L,), jnp.int32)
        for j in range(_NV):
            flag_v[pl.ds(_L * j, _L)] = zeros
        for j in range(2 * N_TOK // _L):
            idx = idx_v[pl.ds(_L * j, _L)]
            idx = jnp.minimum(jnp.maximum(idx, 0), E - 1)
            plsc.store_scatter(flag_v, [idx], ones, mask=ones > 0)
        _J0 = K_SPEC // _L
        iota = lax.iota(jnp.int32, _L)
        carry = jnp.int32(0)
        last = jnp.int32(K_SPEC)
        pos = []
        flags = []
        for j in range(_J0, _NV):
            t = flag_v[pl.ds(_L * j, _L)]
            incl = plsc.cumsum(t)
            pos.append(incl - t + carry)
            flags.append(t)
            carry = carry + jnp.sum(t)
            cand = jnp.where(t > 0, iota + _L * j, -1)
            last = jnp.maximum(last, jnp.max(cand))
        for j in range(_NV):
            ids_v[pl.ds(_L * j, _L)] = jnp.full((_L,), last, jnp.int32)
        for j in range(_J0, _NV):
            plsc.store_scatter(ids_v, [pos[j - _J0]], iota + _L * j,
                               mask=flags[j - _J0] > 0)
        ids_v[pl.ds(E, _L)] = jnp.full((_L,), jnp.maximum(carry, 1),
                                       jnp.int32)
        pltpu.async_copy(ids_v, out_hbm, sem).wait()


def _dispatch(gi01):
    mesh = plsc.VectorSubcoreMesh(core_axis_name="c", subcore_axis_name="s")
    f = pl.kernel(
        _dispatch_body,
        out_type=jax.ShapeDtypeStruct((_OUTW,), jnp.int32),
        mesh=mesh,
        scratch_types=[
            pltpu.VMEM((2 * N_TOK,), jnp.int32),
            pltpu.VMEM((E,), jnp.int32),
            pltpu.VMEM((_OUTW,), jnp.int32),
            pltpu.SemaphoreType.DMA,
        ],
        compiler_params=pltpu.CompilerParams(needs_layout_passes=False),
    )
    return f(gi01)


def _swiglu_update(e, x_ref, gi0_ref, gi1_ref, gw0_ref, gw1_ref,
                   w1_ref, w2_ref, w3_ref, out_ref):
    x = x_ref[...]
    w1 = w1_ref[0]
    w2 = w2_ref[0]
    w3 = w3_ref[0]
    h1 = lax.dot_general(x, w1, (((1,), (1,)), ((), ())),
                         preferred_element_type=_F32)
    h2 = lax.dot_general(x, w2, (((1,), (1,)), ((), ())),
                         preferred_element_type=_F32)
    h = h1 * jax.nn.sigmoid(h1) * h2
    y = lax.dot_general(h, w3, (((1,), (1,)), ((), ())),
                        preferred_element_type=_F32)
    g = (jnp.where(gi0_ref[...] == e, gw0_ref[...], 0.0)
         + jnp.where(gi1_ref[...] == e, gw1_ref[...], 0.0))
    out_ref[...] += g * y


def _spec_body(x_ref, gi0_ref, gi1_ref, gw0_ref, gw1_ref,
               w1_ref, w2_ref, w3_ref, out_ref):
    i = pl.program_id(0)

    @pl.when(i == 0)
    def _():
        out_ref[...] = jnp.zeros_like(out_ref)

    _swiglu_update(i, x_ref, gi0_ref, gi1_ref, gw0_ref, gw1_ref,
                   w1_ref, w2_ref, w3_ref, out_ref)


def _spec_experts(x2d, gi0, gi1, gw0, gw1, w1, w2, w3):
    full = lambda i: (0, 0)
    wmap = lambda i: (i, 0, 0)
    return pl.pallas_call(
        _spec_body,
        grid=(K_SPEC,),
        in_specs=[
            pl.BlockSpec((N_TOK, D), full),
            pl.BlockSpec((N_TOK, 1), full),
            pl.BlockSpec((N_TOK, 1), full),
            pl.BlockSpec((N_TOK, 1), full),
            pl.BlockSpec((N_TOK, 1), full),
            pl.BlockSpec((1, H, D), wmap),
            pl.BlockSpec((1, H, D), wmap),
            pl.BlockSpec((1, D, H), wmap),
        ],
        out_specs=pl.BlockSpec((N_TOK, D), full),
        out_shape=jax.ShapeDtypeStruct((N_TOK, D), _F32),
    )(x2d, gi0, gi1, gw0, gw1, w1, w2, w3)


def _rest_body(ids_ref, x_ref, gi0_ref, gi1_ref, gw0_ref, gw1_ref, prev_ref,
               w1_ref, w2_ref, w3_ref, out_ref):
    i = pl.program_id(0)
    e = ids_ref[i]

    @pl.when(i == 0)
    def _():
        out_ref[...] = prev_ref[...]

    _swiglu_update(e, x_ref, gi0_ref, gi1_ref, gw0_ref, gw1_ref,
                   w1_ref, w2_ref, w3_ref, out_ref)


def _rest_experts(cnt, ids, x2d, gi0, gi1, gw0, gw1, prev, w1, w2, w3):
    full = lambda i, ids_ref: (0, 0)
    wmap = lambda i, ids_ref: (ids_ref[i], 0, 0)
    grid_spec = pltpu.PrefetchScalarGridSpec(
        num_scalar_prefetch=1,
        grid=(cnt,),
        in_specs=[
            pl.BlockSpec((N_TOK, D), full),
            pl.BlockSpec((N_TOK, 1), full),
            pl.BlockSpec((N_TOK, 1), full),
            pl.BlockSpec((N_TOK, 1), full),
            pl.BlockSpec((N_TOK, 1), full),
            pl.BlockSpec((N_TOK, D), full),
            pl.BlockSpec((1, H, D), wmap),
            pl.BlockSpec((1, H, D), wmap),
            pl.BlockSpec((1, D, H), wmap),
        ],
        out_specs=pl.BlockSpec((N_TOK, D), full),
    )
    return pl.pallas_call(
        _rest_body,
        grid_spec=grid_spec,
        out_shape=jax.ShapeDtypeStruct((N_TOK, D), _F32),
    )(ids, x2d, gi0, gi1, gw0, gw1, prev, w1, w2, w3)


def kernel(x, router_w, w1, w2, w3):
    B, T, Dm = x.shape
    x2d = x.reshape(B * T, Dm)
    gi01, gi0, gi1, gw0, gw1, aux = _router(x2d, router_w)
    idscnt = _dispatch(gi01.reshape(2 * N_TOK))
    part = _spec_experts(x2d, gi0, gi1, gw0, gw1, w1, w2, w3)
    out = _rest_experts(idscnt[E], idscnt, x2d, gi0, gi1, gw0, gw1, part,
                        w1, w2, w3)
    return out.reshape(B, T, Dm), aux[0, 0]


# speedup vs baseline: 1.0426x; 1.0426x over previous
import jax
import jax.numpy as jnp
from jax import lax
from jax.experimental import pallas as pl
from jax.experimental.pallas import tpu as pltpu
from jax.experimental.pallas import tpu_sc as plsc

N_TOK = 64
D = 768
E = 64
H = 768

_F32 = jnp.float32


def _router_body(x_ref, rw_ref, gi01_ref, gi0_ref, gi1_ref, gw0_ref, gw1_ref,
                 aux_ref):
    x = x_ref[...]
    rw = rw_ref[...]
    logits = lax.dot_general(x, rw, (((1,), (1,)), ((), ())),
                             preferred_element_type=_F32)
    m = jnp.max(logits, axis=1, keepdims=True)
    el = jnp.exp(logits - m)
    probs = el / jnp.sum(el, axis=1, keepdims=True)

    lane = lax.broadcasted_iota(jnp.int32, (N_TOK, E), 1)
    v0 = jnp.max(probs, axis=1, keepdims=True)
    i0 = jnp.min(jnp.where(probs == v0, lane, E), axis=1, keepdims=True)
    probs2 = jnp.where(lane == i0, -1.0, probs)
    v1 = jnp.max(probs2, axis=1, keepdims=True)
    i1 = jnp.min(jnp.where(probs2 == v1, lane, E), axis=1, keepdims=True)

    s = v0 + v1
    gi01_ref[0:N_TOK] = i0
    gi01_ref[N_TOK:2 * N_TOK] = i1
    gi0_ref[...] = i0
    gi1_ref[...] = i1
    gw0_ref[...] = v0 / s
    gw1_ref[...] = v1 / s

    one0 = (lane == i0).astype(_F32)
    f = jnp.sum(one0, axis=0, keepdims=True) / N_TOK
    P = jnp.sum(probs, axis=0, keepdims=True) / N_TOK
    aux_ref[...] = jnp.sum(f * P, keepdims=True) * E


def _router(x2d, router_w):
    return pl.pallas_call(
        _router_body,
        out_shape=(
            jax.ShapeDtypeStruct((2 * N_TOK, 1), jnp.int32),
            jax.ShapeDtypeStruct((N_TOK, 1), jnp.int32),
            jax.ShapeDtypeStruct((N_TOK, 1), jnp.int32),
            jax.ShapeDtypeStruct((N_TOK, 1), _F32),
            jax.ShapeDtypeStruct((N_TOK, 1), _F32),
            jax.ShapeDtypeStruct((1, 1), _F32),
        ),
    )(x2d, router_w)


_L = 16
_NV = E // _L
_OUTW = E + _L


def _dispatch_body(gi01_hbm, out_hbm, idx_v, flag_v, ids_v, sem):
    cid = lax.axis_index("c")
    sid = lax.axis_index("s")

    @pl.when((cid == 0) & (sid == 0))
    def _():
        pltpu.sync_copy(gi01_hbm, idx_v)
        zeros = jnp.zeros((_L,), jnp.int32)
        ones = jnp.ones((_L,), jnp.int32)
        for j in range(_NV):
            flag_v[pl.ds(_L * j, _L)] = zeros
        for j in range(2 * N_TOK // _L):
            idx = idx_v[pl.ds(_L * j, _L)]
            idx = jnp.minimum(jnp.maximum(idx, 0), E - 1)
            plsc.store_scatter(flag_v, [idx], ones, mask=ones > 0)
        iota = lax.iota(jnp.int32, _L)
        carry = jnp.int32(0)
        last = jnp.int32(0)
        pos = []
        flags = []
        for j in range(_NV):
            t = flag_v[pl.ds(_L * j, _L)]
            incl = plsc.cumsum(t)
            pos.append(incl - t + carry)
            flags.append(t)
            carry = carry + jnp.sum(t)
            cand = jnp.where(t > 0, iota + _L * j, -1)
            last = jnp.maximum(last, jnp.max(cand))
        for j in range(_NV):
            ids_v[pl.ds(_L * j, _L)] = jnp.full((_L,), last, jnp.int32)
        for j in range(_NV):
            plsc.store_scatter(ids_v, [pos[j]], iota + _L * j,
                               mask=flags[j] > 0)
        ids_v[pl.ds(E, _L)] = jnp.full((_L,), carry, jnp.int32)
        pltpu.async_copy(ids_v, out_hbm, sem).wait()


def _dispatch(gi01):
    mesh = plsc.VectorSubcoreMesh(core_axis_name="c", subcore_axis_name="s")
    f = pl.kernel(
        _dispatch_body,
        out_type=jax.ShapeDtypeStruct((_OUTW,), jnp.int32),
        mesh=mesh,
        scratch_types=[
            pltpu.VMEM((2 * N_TOK,), jnp.int32),
            pltpu.VMEM((E,), jnp.int32),
            pltpu.VMEM((_OUTW,), jnp.int32),
            pltpu.SemaphoreType.DMA,
        ],
        compiler_params=pltpu.CompilerParams(needs_layout_passes=False),
    )
    return f(gi01)


def _swiglu_update(e, x_ref, gi0_ref, gi1_ref, gw0_ref, gw1_ref,
                   w1_ref, w2_ref, w3_ref, out_ref):
    x = x_ref[...]
    w1 = w1_ref[0]
    w2 = w2_ref[0]
    w3 = w3_ref[0]
    h1 = lax.dot_general(x, w1, (((1,), (1,)), ((), ())),
                         preferred_element_type=_F32)
    h2 = lax.dot_general(x, w2, (((1,), (1,)), ((), ())),
                         preferred_element_type=_F32)
    h = h1 * jax.nn.sigmoid(h1) * h2
    y = lax.dot_general(h, w3, (((1,), (1,)), ((), ())),
                        preferred_element_type=_F32)
    g = (jnp.where(gi0_ref[...] == e, gw0_ref[...], 0.0)
         + jnp.where(gi1_ref[...] == e, gw1_ref[...], 0.0))
    out_ref[...] += g * y


def _expert_body(ids_ref, x_ref, gi0_ref, gi1_ref, gw0_ref, gw1_ref,
                 w1_ref, w2_ref, w3_ref, out_ref):
    i = pl.program_id(0)

    @pl.when(i == 0)
    def _():
        out_ref[...] = jnp.zeros_like(out_ref)

    _swiglu_update(ids_ref[i], x_ref, gi0_ref, gi1_ref, gw0_ref, gw1_ref,
                   w1_ref, w2_ref, w3_ref, out_ref)


def _experts(cnt, ids, x2d, gi0, gi1, gw0, gw1, w1, w2, w3):
    full = lambda i, ids_ref: (0, 0)
    wmap = lambda i, ids_ref: (ids_ref[i], 0, 0)
    grid_spec = pltpu.PrefetchScalarGridSpec(
        num_scalar_prefetch=1,
        grid=(cnt,),
        in_specs=[
            pl.BlockSpec((N_TOK, D), full),
            pl.BlockSpec((N_TOK, 1), full),
            pl.BlockSpec((N_TOK, 1), full),
            pl.BlockSpec((N_TOK, 1), full),
            pl.BlockSpec((N_TOK, 1), full),
            pl.BlockSpec((1, H, D), wmap),
            pl.BlockSpec((1, H, D), wmap),
            pl.BlockSpec((1, D, H), wmap),
        ],
        out_specs=pl.BlockSpec((N_TOK, D), full),
    )
    return pl.pallas_call(
        _expert_body,
        grid_spec=grid_spec,
        out_shape=jax.ShapeDtypeStruct((N_TOK, D), _F32),
    )(ids, x2d, gi0, gi1, gw0, gw1, w1, w2, w3)


def kernel(x, router_w, w1, w2, w3):
    B, T, Dm = x.shape
    x2d = x.reshape(B * T, Dm)
    gi01, gi0, gi1, gw0, gw1, aux = _router(x2d, router_w)
    idscnt = _dispatch(gi01.reshape(2 * N_TOK))
    out = _experts(idscnt[E], idscnt, x2d, gi0, gi1, gw0, gw1, w1, w2, w3)
    return out.reshape(B, T, Dm), aux[0, 0]
